# baseline (speedup 1.0000x reference)
import jax
import jax.numpy as jnp
from jax.experimental import pallas as pl

_NUM_CLASSES = 6
_PROTO_N_AXIS = 2
_KDIM = 128
_SCALE_BM = 256


def _q_kernel(pr_ref, wq_ref, bq_ref, q_ref):
    for i in range(_NUM_CLASSES):
        p = pr_ref[i]
        w = wq_ref[i]
        q = jax.lax.dot_general(
            p, w, (((1,), (1,)), ((), ())), preferred_element_type=jnp.float32
        )
        q_ref[i] = q + bq_ref[i][None, :]


def _scale_kernel(x_ref, o_ref):
    o_ref[...] = x_ref[...] * jnp.float32(1.0 / 12.0)


def kernel(prototypes, assp_features, DomainTrain, Wk, bk, Wq, bq):
    b, c, h, w = assp_features.shape
    nc = prototypes.shape[1]
    pn = prototypes.shape[2]
    d = prototypes.shape[3]

    pr = jnp.transpose(prototypes, (1, 0, 2, 3)).reshape(nc, b * pn, d)
    q_all = pl.pallas_call(
        _q_kernel,
        out_shape=jax.ShapeDtypeStruct((nc, b * pn, _KDIM), jnp.float32),
    )(pr, Wq, bq)
    qs = q_all.reshape(nc, b, pn, _KDIM)

    rows = b * c
    cols = h * w
    x2 = assp_features.reshape(rows, cols)
    grid = rows // _SCALE_BM
    out2 = pl.pallas_call(
        _scale_kernel,
        grid=(grid,),
        in_specs=[pl.BlockSpec((_SCALE_BM, cols), lambda m: (m, 0))],
        out_specs=pl.BlockSpec((_SCALE_BM, cols), lambda m: (m, 0)),
        out_shape=jax.ShapeDtypeStruct((rows, cols), jnp.float32),
    )(x2)
    assp_weighted = out2.reshape(b, c, h, w)

    protos_out = tuple(qs[i, :, j] for i in range(nc) for j in range(pn))
    return (assp_weighted,) + protos_out


# speedup vs baseline: 1.2853x; 1.2853x over previous
import jax
import jax.numpy as jnp
from jax.experimental import pallas as pl

_NUM_CLASSES = 6
_KDIM = 128
_BC = 128


def _fused_kernel(pr_ref, wq_ref, bq_ref, x_ref, o_ref, q_ref):
    o_ref[...] = x_ref[...] * jnp.float32(1.0 / 12.0)

    b = pl.program_id(0)
    c = pl.program_id(1)

    @pl.when(jnp.logical_and(b == 0, c == 0))
    def _():
        for i in range(_NUM_CLASSES):
            for j in range(2):
                p = pr_ref[:, i, j, :]
                q = jax.lax.dot_general(
                    p, wq_ref[i], (((1,), (1,)), ((), ())),
                    preferred_element_type=jnp.float32,
                )
                q_ref[i * 2 + j] = q + bq_ref[i][None, :]


def kernel(prototypes, assp_features, DomainTrain, Wk, bk, Wq, bq):
    b, c, h, w = assp_features.shape
    nc = prototypes.shape[1]
    pn = prototypes.shape[2]
    npairs = nc * pn

    grid = (b, c // _BC)
    out, q_all = pl.pallas_call(
        _fused_kernel,
        grid=grid,
        in_specs=[
            pl.BlockSpec(prototypes.shape, lambda bi, ci: (0, 0, 0, 0)),
            pl.BlockSpec(Wq.shape, lambda bi, ci: (0, 0, 0)),
            pl.BlockSpec(bq.shape, lambda bi, ci: (0, 0)),
            pl.BlockSpec((1, _BC, h, w), lambda bi, ci: (bi, ci, 0, 0)),
        ],
        out_specs=[
            pl.BlockSpec((1, _BC, h, w), lambda bi, ci: (bi, ci, 0, 0)),
            pl.BlockSpec((npairs, b, _KDIM), lambda bi, ci: (0, 0, 0)),
        ],
        out_shape=[
            jax.ShapeDtypeStruct((b, c, h, w), jnp.float32),
            jax.ShapeDtypeStruct((npairs, b, _KDIM), jnp.float32),
        ],
    )(prototypes, Wq, bq, assp_features)

    return (out,) + tuple(q_all[p] for p in range(npairs))
